# baseline (speedup 1.0000x reference)
import functools

import jax
import jax.numpy as jnp
import numpy as np
from jax.experimental import pallas as pl
from jax.experimental.pallas import tpu as pltpu

PIMG = 40
TM = 8


def _pack_convt(up_w, W1, C2):
    Cin, Cup = up_w.shape[0], up_w.shape[1]
    A = jnp.transpose(up_w.astype(jnp.float32), (2, 0, 3, 1))
    A = jnp.concatenate([jnp.zeros((2, Cin, 2, C2), jnp.float32), A], axis=3)
    A = A.reshape(2, Cin, 2 * (C2 + Cup))
    eye = jnp.eye(W1, dtype=jnp.float32)
    Wt = jnp.einsum("wu,hij->hiwuj", eye, A)
    return Wt.reshape(2, Cin * W1, W1 * 2 * (C2 + Cup))


def _band_windows(w_oihw, GW):
    Cin, Cout = w_oihw.shape[1], w_oihw.shape[0]
    Wk = jnp.transpose(w_oihw.astype(jnp.float32), (2, 3, 1, 0))
    mats = []
    for kh in range(3):
        m = None
        for kw in range(3):
            S = np.zeros((3 * GW, GW), np.float32)
            for w in range(GW):
                S[w + kw + GW - 1, w] = 1.0
            t = jnp.einsum("vw,io->viwo", jnp.asarray(S), Wk[kh, kw])
            m = t if m is None else m + t
        mats.append(m.reshape(3 * GW * Cin, GW * Cout))
    Bg = jnp.stack(mats)
    LB = GW * Cin
    mL = jnp.ones((3 * GW * Cin, 1), jnp.float32).at[:LB].set(0.0)
    mR = jnp.ones((3 * GW * Cin, 1), jnp.float32).at[2 * LB:].set(0.0)
    return jnp.stack([Bg * mL, Bg, Bg * mR, Bg * mL * mR], axis=1)


def _interleave_mats(NB, H1):
    H2 = 2 * H1
    R = np.zeros((2, NB * H2, NB * H1), np.float32)
    for i in range(NB):
        for h in range(H1):
            R[0, i * H2 + 2 * h, i * H1 + h] = 1.0
            R[1, i * H2 + 2 * h + 1, i * H1 + h] = 1.0
    return jnp.asarray(R, dtype=jnp.bfloat16)


def _bn_affine(sum_lane, ssq_lane, count, C, gamma, beta, eps):
    s = jnp.sum(sum_lane.reshape(-1, C), axis=0)
    ss = jnp.sum(ssq_lane.reshape(-1, C), axis=0)
    mean = s / count
    var = ss / count - mean * mean
    scale = gamma / jnp.sqrt(var + eps)
    shift = beta - mean * scale
    return scale, shift


def _banded_windowed_matmuls(xm, x0, xp, b_ref, LB, NG):
    groups = []
    for j in range(NG):
        sl = slice(j * LB, (j + 3) * LB)
        v = 1 if 0 < j < NG - 1 else (3 if NG == 1 else (0 if j == 0 else 2))
        a = jnp.dot(xm[:, sl], b_ref[0, v], preferred_element_type=jnp.float32)
        a = a + jnp.dot(x0[:, sl], b_ref[1, v], preferred_element_type=jnp.float32)
        a = a + jnp.dot(xp[:, sl], b_ref[2, v], preferred_element_type=jnp.float32)
        groups.append(a)
    return jnp.concatenate(groups, axis=1) if len(groups) > 1 else groups[0]


def _k1_body(x1_ref, x2f_ref, wt_ref, btw_ref, r0_ref, r1_ref, e_ref, b1_ref,
             o_ref, s_ref, ss_ref, xcat_ref, xm_ref, xp_ref,
             *, NB, H1, H2, LB, NG):
    M = NB * PIMG
    WL = xcat_ref.shape[1] - 2 * LB

    xcat_ref[...] = jnp.zeros_like(xcat_ref)

    x1 = jnp.transpose(x1_ref[...], (0, 2, 1, 3)).astype(jnp.bfloat16)
    x1 = x1.reshape(NB * H1, x1.shape[2] * x1.shape[3])
    x2 = jnp.transpose(x2f_ref[...], (0, 2, 1, 3)).astype(jnp.bfloat16)
    x2 = x2.reshape(NB * H2, x2.shape[2] * x2.shape[3])

    bt = btw_ref[...]
    y0 = (jnp.dot(x1, wt_ref[0], preferred_element_type=jnp.float32)
          + bt).astype(jnp.bfloat16)
    y1 = (jnp.dot(x1, wt_ref[1], preferred_element_type=jnp.float32)
          + bt).astype(jnp.bfloat16)

    inter = (jnp.dot(r0_ref[...], y0, preferred_element_type=jnp.float32)
             + jnp.dot(r1_ref[...], y1, preferred_element_type=jnp.float32))

    x2e = jnp.dot(x2, e_ref[...], preferred_element_type=jnp.float32)
    data = x2e + inter
    for i in range(NB):
        xcat_ref[pl.ds(TM + i * PIMG, H2), LB:LB + WL] = (
            data[i * H2:(i + 1) * H2].astype(xcat_ref.dtype))

    xm_ref[...] = xcat_ref[TM - 1:TM - 1 + M, :]
    xp_ref[...] = xcat_ref[TM + 1:TM + 1 + M, :]

    acc = _banded_windowed_matmuls(xm_ref[...], xcat_ref[TM:TM + M, :],
                                   xp_ref[...], b1_ref, LB, NG)

    rr = jax.lax.broadcasted_iota(jnp.int32, (M, 1), 0) % PIMG
    mask = (rr < H2).astype(jnp.float32)
    accm = acc * mask
    s_ref[...] = jnp.sum(accm, axis=0, keepdims=True)
    ss_ref[...] = jnp.sum(accm * acc, axis=0, keepdims=True)

    o_ref[...] = acc.astype(o_ref.dtype)


def _k2_body(h_ref, b2_ref, sc_ref, sh_ref,
             o_ref, s_ref, ss_ref, xact_ref, xm_ref, xp_ref, *, NB, H2, LB, NG):
    M = NB * PIMG
    Mx = xact_ref.shape[0]
    WL = xact_ref.shape[1] - 2 * LB

    xact_ref[0:TM, :] = jnp.zeros((TM, xact_ref.shape[1]), xact_ref.dtype)
    xact_ref[Mx - TM:Mx, :] = jnp.zeros((TM, xact_ref.shape[1]), xact_ref.dtype)
    xact_ref[:, 0:LB] = jnp.zeros((Mx, LB), xact_ref.dtype)
    xact_ref[:, LB + WL:] = jnp.zeros((Mx, LB), xact_ref.dtype)

    rr = jax.lax.broadcasted_iota(jnp.int32, (M, 1), 0) % PIMG
    valid = rr < H2
    val = jnp.maximum(h_ref[...].astype(jnp.float32) * sc_ref[...] + sh_ref[...],
                      0.0)
    xact_ref[pl.ds(TM, M), LB:LB + WL] = jnp.where(valid, val, 0.0).astype(
        xact_ref.dtype)

    xm_ref[...] = xact_ref[TM - 1:TM - 1 + M, :]
    xp_ref[...] = xact_ref[TM + 1:TM + 1 + M, :]

    acc = _banded_windowed_matmuls(xm_ref[...], xact_ref[TM:TM + M, :],
                                   xp_ref[...], b2_ref, LB, NG)

    mask = valid.astype(jnp.float32)
    accm = acc * mask
    s_ref[...] = jnp.sum(accm, axis=0, keepdims=True)
    ss_ref[...] = jnp.sum(accm * acc, axis=0, keepdims=True)

    for i in range(NB):
        o_ref[i, :, :] = acc[i * PIMG:i * PIMG + H2].astype(o_ref.dtype)


def _k3_body(x_ref, p_ref, sc_ref, sh_ref, o_ref, *, H2, W2):
    xp = jnp.dot(x_ref[...], p_ref[...], preferred_element_type=jnp.float32)
    y = jnp.maximum(xp * sc_ref[...] + sh_ref[...], 0.0)
    NBk, Cout = o_ref.shape[0], o_ref.shape[1]
    y4 = y.reshape(NBk, H2, Cout, W2)
    o_ref[...] = jnp.transpose(y4, (0, 2, 1, 3))


def kernel(up_w, up_b, conv1_w, conv2_w, bn1_gamma, bn1_beta,
           bn2_gamma, bn2_beta, x1_nchw, x2_nchw):
    eps = 1e-5
    N, Cin, H1, W1 = x1_nchw.shape
    _, C2, H2, W2 = x2_nchw.shape
    Cup = up_w.shape[1]
    Cmid = conv1_w.shape[0]
    Cout = conv2_w.shape[0]
    Ctot = C2 + Cup
    assert H2 == 2 * H1 and W2 == 2 * W1
    assert conv1_w.shape[1] == Ctot
    assert H2 < PIMG and PIMG % 8 == 0

    bf = jnp.bfloat16

    GW1 = min(W2, max(1, 128 // Ctot))
    GW2 = min(W2, max(1, 128 // Cmid))
    assert W2 % GW1 == 0 and W2 % GW2 == 0
    LB1, LBo1, NG1 = GW1 * Ctot, GW1 * Cmid, W2 // GW1
    LB2, LBo2, NG2 = GW2 * Cmid, GW2 * Cout, W2 // GW2

    Wt2 = _pack_convt(up_w, W1, C2).astype(bf)
    btw = jnp.tile(jnp.concatenate([jnp.zeros((C2,), jnp.float32),
                                    up_b.astype(jnp.float32)]), W2)[None, :]
    B1w = _band_windows(conv1_w, GW1).astype(bf)
    B2w = _band_windows(conv2_w, GW2).astype(bf)
    Ex = np.zeros((C2 * W2, W2 * Ctot), np.float32)
    for w in range(W2):
        for c in range(C2):
            Ex[c * W2 + w, w * Ctot + c] = 1.0
    Ex = jnp.asarray(Ex, dtype=bf)

    WCtot = W2 * Ctot
    WCmid = W2 * Cmid
    WCout = W2 * Cout

    NB = 16 if N % 16 == 0 else (8 if N % 8 == 0 else 1)
    G = N // NB
    M = NB * PIMG
    Mx = M + 2 * TM
    Rm = _interleave_mats(NB, H1)

    k1 = functools.partial(_k1_body, NB=NB, H1=H1, H2=H2, LB=LB1, NG=NG1)
    h1, s1, ss1 = pl.pallas_call(
        k1,
        out_shape=(
            jax.ShapeDtypeStruct((G * M, WCmid), bf),
            jax.ShapeDtypeStruct((G, 1, WCmid), jnp.float32),
            jax.ShapeDtypeStruct((G, 1, WCmid), jnp.float32),
        ),
        grid=(G,),
        in_specs=[
            pl.BlockSpec((NB, Cin, H1, W1), lambda g: (g, 0, 0, 0)),
            pl.BlockSpec((NB, C2, H2, W2), lambda g: (g, 0, 0, 0)),
            pl.BlockSpec((2, Cin * W1, WCtot), lambda g: (0, 0, 0)),
            pl.BlockSpec((1, WCtot), lambda g: (0, 0)),
            pl.BlockSpec((NB * H2, NB * H1), lambda g: (0, 0)),
            pl.BlockSpec((NB * H2, NB * H1), lambda g: (0, 0)),
            pl.BlockSpec((C2 * W2, WCtot), lambda g: (0, 0)),
            pl.BlockSpec((3, 4, 3 * LB1, LBo1), lambda g: (0, 0, 0, 0)),
        ],
        out_specs=(
            pl.BlockSpec((M, WCmid), lambda g: (g, 0)),
            pl.BlockSpec((None, 1, WCmid), lambda g: (g, 0, 0)),
            pl.BlockSpec((None, 1, WCmid), lambda g: (g, 0, 0)),
        ),
        scratch_shapes=[pltpu.VMEM((Mx, WCtot + 2 * LB1), bf),
                        pltpu.VMEM((M, WCtot + 2 * LB1), bf),
                        pltpu.VMEM((M, WCtot + 2 * LB1), bf)],
        compiler_params=pltpu.CompilerParams(dimension_semantics=("parallel",)),
    )(x1_nchw.astype(jnp.float32), x2_nchw.astype(jnp.float32),
      Wt2, btw, Rm[0], Rm[1], Ex, B1w)

    scale1, shift1 = _bn_affine(s1, ss1, N * H2 * W2, Cmid, bn1_gamma, bn1_beta, eps)

    k2 = functools.partial(_k2_body, NB=NB, H2=H2, LB=LB2, NG=NG2)
    h2, s2, ss2 = pl.pallas_call(
        k2,
        out_shape=(
            jax.ShapeDtypeStruct((N, H2, WCout), bf),
            jax.ShapeDtypeStruct((G, 1, WCout), jnp.float32),
            jax.ShapeDtypeStruct((G, 1, WCout), jnp.float32),
        ),
        grid=(G,),
        in_specs=[
            pl.BlockSpec((M, WCmid), lambda g: (g, 0)),
            pl.BlockSpec((3, 4, 3 * LB2, LBo2), lambda g: (0, 0, 0, 0)),
            pl.BlockSpec((1, WCmid), lambda g: (0, 0)),
            pl.BlockSpec((1, WCmid), lambda g: (0, 0)),
        ],
        out_specs=(
            pl.BlockSpec((NB, H2, WCout), lambda g: (g, 0, 0)),
            pl.BlockSpec((None, 1, WCout), lambda g: (g, 0, 0)),
            pl.BlockSpec((None, 1, WCout), lambda g: (g, 0, 0)),
        ),
        scratch_shapes=[pltpu.VMEM((Mx, WCmid + 2 * LB2), bf),
                        pltpu.VMEM((M, WCmid + 2 * LB2), bf),
                        pltpu.VMEM((M, WCmid + 2 * LB2), bf)],
        compiler_params=pltpu.CompilerParams(dimension_semantics=("parallel",)),
    )(h1, B2w, jnp.tile(scale1, W2)[None, :], jnp.tile(shift1, W2)[None, :])

    scale2, shift2 = _bn_affine(s2, ss2, N * H2 * W2, Cout, bn2_gamma, bn2_beta, eps)

    Pcw = np.zeros((WCout, WCout), np.float32)
    for w in range(W2):
        for c in range(Cout):
            Pcw[w * Cout + c, c * W2 + w] = 1.0
    Pcw = jnp.asarray(Pcw, dtype=bf)

    Mrows = N * H2
    NBk = 16 if N % 16 == 0 else (8 if N % 8 == 0 else 1)
    k3 = functools.partial(_k3_body, H2=H2, W2=W2)
    out = pl.pallas_call(
        k3,
        out_shape=jax.ShapeDtypeStruct((N, Cout, H2, W2), jnp.float32),
        grid=(N // NBk,),
        in_specs=[
            pl.BlockSpec((NBk * H2, WCout), lambda i: (i, 0)),
            pl.BlockSpec((WCout, WCout), lambda i: (0, 0)),
            pl.BlockSpec((1, WCout), lambda i: (0, 0)),
            pl.BlockSpec((1, WCout), lambda i: (0, 0)),
        ],
        out_specs=pl.BlockSpec((NBk, Cout, H2, W2), lambda i: (i, 0, 0, 0)),
        compiler_params=pltpu.CompilerParams(dimension_semantics=("parallel",)),
    )(h2.reshape(Mrows, WCout), Pcw, jnp.repeat(scale2, W2)[None, :],
      jnp.repeat(shift2, W2)[None, :])
    return out


# speedup vs baseline: 1.1904x; 1.1904x over previous
import functools

import jax
import jax.numpy as jnp
import numpy as np
from jax.experimental import pallas as pl
from jax.experimental.pallas import tpu as pltpu

PIMG = 40
TM = 8


def _pack_convt(up_w, W1, C2):
    Cin, Cup = up_w.shape[0], up_w.shape[1]
    A = jnp.transpose(up_w.astype(jnp.float32), (2, 0, 3, 1))
    A = jnp.concatenate([jnp.zeros((2, Cin, 2, C2), jnp.float32), A], axis=3)
    A = A.reshape(2, Cin, 2 * (C2 + Cup))
    eye = jnp.eye(W1, dtype=jnp.float32)
    Wt = jnp.einsum("wu,hij->hwiuj", eye, A)
    return Wt.reshape(2, W1 * Cin, W1 * 2 * (C2 + Cup))


def _band_windows(w_oihw, GW):
    Cin, Cout = w_oihw.shape[1], w_oihw.shape[0]
    Wk = jnp.transpose(w_oihw.astype(jnp.float32), (2, 3, 1, 0))
    mats = []
    for kh in range(3):
        m = None
        for kw in range(3):
            S = np.zeros((3 * GW, GW), np.float32)
            for w in range(GW):
                S[w + kw + GW - 1, w] = 1.0
            t = jnp.einsum("vw,io->viwo", jnp.asarray(S), Wk[kh, kw])
            m = t if m is None else m + t
        mats.append(m.reshape(3 * GW * Cin, GW * Cout))
    Bg = jnp.stack(mats)
    LB = GW * Cin
    mL = jnp.ones((3 * GW * Cin, 1), jnp.float32).at[:LB].set(0.0)
    mR = jnp.ones((3 * GW * Cin, 1), jnp.float32).at[2 * LB:].set(0.0)
    return jnp.stack([Bg * mL, Bg, Bg * mR, Bg * mL * mR], axis=1)


def _interleave_mats(NB, H1):
    H2 = 2 * H1
    R = np.zeros((2, NB * H2, NB * H1), np.float32)
    for i in range(NB):
        for h in range(H1):
            R[0, i * H2 + 2 * h, i * H1 + h] = 1.0
            R[1, i * H2 + 2 * h + 1, i * H1 + h] = 1.0
    return jnp.asarray(R, dtype=jnp.bfloat16)


def _bn_affine(sum_lane, ssq_lane, count, C, gamma, beta, eps):
    s = jnp.sum(sum_lane.reshape(-1, C), axis=0)
    ss = jnp.sum(ssq_lane.reshape(-1, C), axis=0)
    mean = s / count
    var = ss / count - mean * mean
    scale = gamma / jnp.sqrt(var + eps)
    shift = beta - mean * scale
    return scale, shift


def _banded_windowed_matmuls(xm, x0, xp, b_ref, LB, NG):
    groups = []
    for j in range(NG):
        sl = slice(j * LB, (j + 3) * LB)
        v = 1 if 0 < j < NG - 1 else (3 if NG == 1 else (0 if j == 0 else 2))
        a = jnp.dot(xm[:, sl], b_ref[0, v], preferred_element_type=jnp.float32)
        a = a + jnp.dot(x0[:, sl], b_ref[1, v], preferred_element_type=jnp.float32)
        a = a + jnp.dot(xp[:, sl], b_ref[2, v], preferred_element_type=jnp.float32)
        groups.append(a)
    return jnp.concatenate(groups, axis=1) if len(groups) > 1 else groups[0]


def _k1_body(x1_ref, x2f_ref, wt_ref, btw_ref, r0_ref, r1_ref, e_ref, b1_ref,
             o_ref, s_ref, ss_ref, xcat_ref, xm_ref, xp_ref,
             *, NB, H1, H2, LB, NG, LBu, NGu):
    M = NB * PIMG
    WL = xcat_ref.shape[1] - 2 * LB
    UO2 = wt_ref.shape[1]
    UOg = UO2 // 2

    xcat_ref[...] = jnp.zeros_like(xcat_ref)

    x1 = x1_ref[...].reshape(NB * H1, NGu * LBu)
    bt = btw_ref[...]
    y0s, y1s = [], []
    for j in range(NGu):
        r = jnp.dot(x1[:, j * LBu:(j + 1) * LBu], wt_ref[...],
                    preferred_element_type=jnp.float32)
        y0s.append(r[:, :UOg])
        y1s.append(r[:, UOg:])
    y0 = (jnp.concatenate(y0s, axis=1) if NGu > 1 else y0s[0]) + bt
    y1 = (jnp.concatenate(y1s, axis=1) if NGu > 1 else y1s[0]) + bt
    y0 = y0.astype(jnp.bfloat16)
    y1 = y1.astype(jnp.bfloat16)

    inter = (jnp.dot(r0_ref[...], y0, preferred_element_type=jnp.float32)
             + jnp.dot(r1_ref[...], y1, preferred_element_type=jnp.float32))

    x2e = jnp.dot(x2f_ref[...].reshape(NB * H2, e_ref.shape[0]), e_ref[...],
                  preferred_element_type=jnp.float32)
    data = x2e + inter
    for i in range(NB):
        xcat_ref[pl.ds(TM + i * PIMG, H2), LB:LB + WL] = (
            data[i * H2:(i + 1) * H2].astype(xcat_ref.dtype))

    xm_ref[...] = xcat_ref[TM - 1:TM - 1 + M, :]
    xp_ref[...] = xcat_ref[TM + 1:TM + 1 + M, :]

    acc = _banded_windowed_matmuls(xm_ref[...], xcat_ref[TM:TM + M, :],
                                   xp_ref[...], b1_ref, LB, NG)

    rr = jax.lax.broadcasted_iota(jnp.int32, (M, 1), 0) % PIMG
    mask = (rr < H2).astype(jnp.float32)
    accm = acc * mask
    s_ref[...] = jnp.sum(accm, axis=0, keepdims=True)
    ss_ref[...] = jnp.sum(accm * acc, axis=0, keepdims=True)

    o_ref[...] = acc.astype(o_ref.dtype)


def _k2_body(h_ref, b2_ref, sc_ref, sh_ref,
             o_ref, s_ref, ss_ref, xact_ref, xm_ref, xp_ref, *, NB, H2, LB, NG):
    M = NB * PIMG
    Mx = xact_ref.shape[0]
    WL = xact_ref.shape[1] - 2 * LB

    xact_ref[0:TM, :] = jnp.zeros((TM, xact_ref.shape[1]), xact_ref.dtype)
    xact_ref[Mx - TM:Mx, :] = jnp.zeros((TM, xact_ref.shape[1]), xact_ref.dtype)
    xact_ref[:, 0:LB] = jnp.zeros((Mx, LB), xact_ref.dtype)
    xact_ref[:, LB + WL:] = jnp.zeros((Mx, LB), xact_ref.dtype)

    rr = jax.lax.broadcasted_iota(jnp.int32, (M, 1), 0) % PIMG
    valid = rr < H2
    val = jnp.maximum(h_ref[...].astype(jnp.float32) * sc_ref[...] + sh_ref[...],
                      0.0)
    xact_ref[pl.ds(TM, M), LB:LB + WL] = jnp.where(valid, val, 0.0).astype(
        xact_ref.dtype)

    xm_ref[...] = xact_ref[TM - 1:TM - 1 + M, :]
    xp_ref[...] = xact_ref[TM + 1:TM + 1 + M, :]

    acc = _banded_windowed_matmuls(xm_ref[...], xact_ref[TM:TM + M, :],
                                   xp_ref[...], b2_ref, LB, NG)

    mask = valid.astype(jnp.float32)
    accm = acc * mask
    s_ref[...] = jnp.sum(accm, axis=0, keepdims=True)
    ss_ref[...] = jnp.sum(accm * acc, axis=0, keepdims=True)

    for i in range(NB):
        o_ref[i, :, :] = acc[i * PIMG:i * PIMG + H2].astype(o_ref.dtype)


def _k3_body(x_ref, p_ref, sc_ref, sh_ref, o_ref, *, H2, W2):
    xp = jnp.dot(x_ref[...], p_ref[...], preferred_element_type=jnp.float32)
    y = jnp.maximum(xp * sc_ref[...] + sh_ref[...], 0.0)
    NBk, Cout = o_ref.shape[0], o_ref.shape[1]
    y4 = y.reshape(NBk, H2, Cout, W2)
    o_ref[...] = jnp.transpose(y4, (0, 2, 1, 3))


def kernel(up_w, up_b, conv1_w, conv2_w, bn1_gamma, bn1_beta,
           bn2_gamma, bn2_beta, x1_nchw, x2_nchw):
    eps = 1e-5
    x1 = jnp.transpose(x1_nchw, (0, 2, 3, 1)).astype(jnp.float32)
    x2 = jnp.transpose(x2_nchw, (0, 2, 3, 1)).astype(jnp.float32)
    N, H1, W1, Cin = x1.shape
    _, H2, W2, C2 = x2.shape
    Cup = up_w.shape[1]
    Cmid = conv1_w.shape[0]
    Cout = conv2_w.shape[0]
    Ctot = C2 + Cup
    assert H2 == 2 * H1 and W2 == 2 * W1
    assert conv1_w.shape[1] == Ctot
    assert H2 < PIMG and PIMG % 8 == 0

    bf = jnp.bfloat16
    x1f = x1.reshape(N, H1, W1 * Cin).astype(bf)
    x2f = x2.reshape(N, H2, W2 * C2).astype(bf)

    GW1 = min(W2, max(1, 128 // Ctot))
    GW2 = min(W2, max(1, 128 // Cmid))
    GWu = min(W1, max(1, 128 // Cin))
    assert W2 % GW1 == 0 and W2 % GW2 == 0 and W1 % GWu == 0
    LB1, LBo1, NG1 = GW1 * Ctot, GW1 * Cmid, W2 // GW1
    LB2, LBo2, NG2 = GW2 * Cmid, GW2 * Cout, W2 // GW2
    LBu, UOg, NGu = GWu * Cin, 2 * GWu * Ctot, W1 // GWu

    Wt = _pack_convt(up_w, GWu, C2)
    Wt2 = jnp.concatenate([Wt[0], Wt[1]], axis=1).astype(bf)
    btw = jnp.tile(jnp.concatenate([jnp.zeros((C2,), jnp.float32),
                                    up_b.astype(jnp.float32)]), W2)[None, :]
    B1w = _band_windows(conv1_w, GW1).astype(bf)
    B2w = _band_windows(conv2_w, GW2).astype(bf)
    Ex = np.zeros((W2 * C2, W2 * Ctot), np.float32)
    for w in range(W2):
        for c in range(C2):
            Ex[w * C2 + c, w * Ctot + c] = 1.0
    Ex = jnp.asarray(Ex, dtype=bf)

    W1K = W1 * Cin
    WCtot = W2 * Ctot
    WCmid = W2 * Cmid
    WCout = W2 * Cout

    NB = 32 if N % 32 == 0 else (16 if N % 16 == 0 else (8 if N % 8 == 0 else 1))
    G = N // NB
    M = NB * PIMG
    Mx = M + 2 * TM
    Rm = _interleave_mats(NB, H1)

    k1 = functools.partial(_k1_body, NB=NB, H1=H1, H2=H2,
                           LB=LB1, NG=NG1, LBu=LBu, NGu=NGu)
    h1, s1, ss1 = pl.pallas_call(
        k1,
        out_shape=(
            jax.ShapeDtypeStruct((G * M, WCmid), bf),
            jax.ShapeDtypeStruct((G, 1, WCmid), jnp.float32),
            jax.ShapeDtypeStruct((G, 1, WCmid), jnp.float32),
        ),
        grid=(G,),
        in_specs=[
            pl.BlockSpec((NB, H1, W1K), lambda g: (g, 0, 0)),
            pl.BlockSpec((NB, H2, W2 * C2), lambda g: (g, 0, 0)),
            pl.BlockSpec((LBu, 2 * UOg), lambda g: (0, 0)),
            pl.BlockSpec((1, WCtot), lambda g: (0, 0)),
            pl.BlockSpec((NB * H2, NB * H1), lambda g: (0, 0)),
            pl.BlockSpec((NB * H2, NB * H1), lambda g: (0, 0)),
            pl.BlockSpec((W2 * C2, WCtot), lambda g: (0, 0)),
            pl.BlockSpec((3, 4, 3 * LB1, LBo1), lambda g: (0, 0, 0, 0)),
        ],
        out_specs=(
            pl.BlockSpec((M, WCmid), lambda g: (g, 0)),
            pl.BlockSpec((None, 1, WCmid), lambda g: (g, 0, 0)),
            pl.BlockSpec((None, 1, WCmid), lambda g: (g, 0, 0)),
        ),
        scratch_shapes=[pltpu.VMEM((Mx, WCtot + 2 * LB1), bf),
                        pltpu.VMEM((M, WCtot + 2 * LB1), bf),
                        pltpu.VMEM((M, WCtot + 2 * LB1), bf)],
        compiler_params=pltpu.CompilerParams(dimension_semantics=("parallel",)),
    )(x1f, x2f, Wt2, btw, Rm[0], Rm[1], Ex, B1w)

    scale1, shift1 = _bn_affine(s1, ss1, N * H2 * W2, Cmid, bn1_gamma, bn1_beta, eps)

    k2 = functools.partial(_k2_body, NB=NB, H2=H2, LB=LB2, NG=NG2)
    h2, s2, ss2 = pl.pallas_call(
        k2,
        out_shape=(
            jax.ShapeDtypeStruct((N, H2, WCout), bf),
            jax.ShapeDtypeStruct((G, 1, WCout), jnp.float32),
            jax.ShapeDtypeStruct((G, 1, WCout), jnp.float32),
        ),
        grid=(G,),
        in_specs=[
            pl.BlockSpec((M, WCmid), lambda g: (g, 0)),
            pl.BlockSpec((3, 4, 3 * LB2, LBo2), lambda g: (0, 0, 0, 0)),
            pl.BlockSpec((1, WCmid), lambda g: (0, 0)),
            pl.BlockSpec((1, WCmid), lambda g: (0, 0)),
        ],
        out_specs=(
            pl.BlockSpec((NB, H2, WCout), lambda g: (g, 0, 0)),
            pl.BlockSpec((None, 1, WCout), lambda g: (g, 0, 0)),
            pl.BlockSpec((None, 1, WCout), lambda g: (g, 0, 0)),
        ),
        scratch_shapes=[pltpu.VMEM((Mx, WCmid + 2 * LB2), bf),
                        pltpu.VMEM((M, WCmid + 2 * LB2), bf),
                        pltpu.VMEM((M, WCmid + 2 * LB2), bf)],
        compiler_params=pltpu.CompilerParams(dimension_semantics=("parallel",)),
    )(h1, B2w, jnp.tile(scale1, W2)[None, :], jnp.tile(shift1, W2)[None, :])

    scale2, shift2 = _bn_affine(s2, ss2, N * H2 * W2, Cout, bn2_gamma, bn2_beta, eps)

    Pcw = np.zeros((WCout, WCout), np.float32)
    for w in range(W2):
        for c in range(Cout):
            Pcw[w * Cout + c, c * W2 + w] = 1.0
    Pcw = jnp.asarray(Pcw, dtype=bf)

    Mrows = N * H2
    NBk = 32 if N % 32 == 0 else (16 if N % 16 == 0 else (8 if N % 8 == 0 else 1))
    k3 = functools.partial(_k3_body, H2=H2, W2=W2)
    out = pl.pallas_call(
        k3,
        out_shape=jax.ShapeDtypeStruct((N, Cout, H2, W2), jnp.float32),
        grid=(N // NBk,),
        in_specs=[
            pl.BlockSpec((NBk * H2, WCout), lambda i: (i, 0)),
            pl.BlockSpec((WCout, WCout), lambda i: (0, 0)),
            pl.BlockSpec((1, WCout), lambda i: (0, 0)),
            pl.BlockSpec((1, WCout), lambda i: (0, 0)),
        ],
        out_specs=pl.BlockSpec((NBk, Cout, H2, W2), lambda i: (i, 0, 0, 0)),
        compiler_params=pltpu.CompilerParams(dimension_semantics=("parallel",)),
    )(h2.reshape(Mrows, WCout), Pcw, jnp.repeat(scale2, W2)[None, :],
      jnp.repeat(shift2, W2)[None, :])
    return out


# speedup vs baseline: 1.2300x; 1.0332x over previous
import functools

import jax
import jax.numpy as jnp
import numpy as np
from jax.experimental import pallas as pl
from jax.experimental.pallas import tpu as pltpu

PIMG = 40
TM = 8


def _pack_convt(up_w, W1, C2):
    Cin, Cup = up_w.shape[0], up_w.shape[1]
    A = jnp.transpose(up_w.astype(jnp.float32), (2, 0, 3, 1))
    A = jnp.concatenate([jnp.zeros((2, Cin, 2, C2), jnp.float32), A], axis=3)
    A = A.reshape(2, Cin, 2 * (C2 + Cup))
    eye = jnp.eye(W1, dtype=jnp.float32)
    Wt = jnp.einsum("wu,hij->hwiuj", eye, A)
    return Wt.reshape(2, W1 * Cin, W1 * 2 * (C2 + Cup))


def _band_windows(w_oihw, GW):
    Cin, Cout = w_oihw.shape[1], w_oihw.shape[0]
    Wk = jnp.transpose(w_oihw.astype(jnp.float32), (2, 3, 1, 0))
    mats = []
    for kh in range(3):
        m = None
        for kw in range(3):
            S = np.zeros((3 * GW, GW), np.float32)
            for w in range(GW):
                S[w + kw + GW - 1, w] = 1.0
            t = jnp.einsum("vw,io->viwo", jnp.asarray(S), Wk[kh, kw])
            m = t if m is None else m + t
        mats.append(m.reshape(3 * GW * Cin, GW * Cout))
    Bg = jnp.stack(mats)
    LB = GW * Cin
    mL = jnp.ones((3 * GW * Cin, 1), jnp.float32).at[:LB].set(0.0)
    mR = jnp.ones((3 * GW * Cin, 1), jnp.float32).at[2 * LB:].set(0.0)
    return jnp.stack([Bg * mL, Bg, Bg * mR, Bg * mL * mR], axis=1)


def _interleave_mats(NB, H1):
    H2 = 2 * H1
    R = np.zeros((2, NB * H2, NB * H1), np.float32)
    for i in range(NB):
        for h in range(H1):
            R[0, i * H2 + 2 * h, i * H1 + h] = 1.0
            R[1, i * H2 + 2 * h + 1, i * H1 + h] = 1.0
    return jnp.asarray(R, dtype=jnp.bfloat16)


def _bn_affine(sum_lane, ssq_lane, count, C, gamma, beta, eps):
    s = jnp.sum(sum_lane.reshape(-1, C), axis=0)
    ss = jnp.sum(ssq_lane.reshape(-1, C), axis=0)
    mean = s / count
    var = ss / count - mean * mean
    scale = gamma / jnp.sqrt(var + eps)
    shift = beta - mean * scale
    return scale, shift


def _banded_windowed_matmuls(xm, x0, xp, b_ref, LB, NG):
    groups = []
    for j in range(NG):
        sl = slice(j * LB, (j + 3) * LB)
        v = 1 if 0 < j < NG - 1 else (3 if NG == 1 else (0 if j == 0 else 2))
        a = jnp.dot(xm[:, sl], b_ref[0, v], preferred_element_type=jnp.float32)
        a = a + jnp.dot(x0[:, sl], b_ref[1, v], preferred_element_type=jnp.float32)
        a = a + jnp.dot(xp[:, sl], b_ref[2, v], preferred_element_type=jnp.float32)
        groups.append(a)
    return jnp.concatenate(groups, axis=1) if len(groups) > 1 else groups[0]


def _k1_body(x1_ref, x2f_ref, wt_ref, btw_ref, r0_ref, r1_ref, e_ref, b1_ref,
             o_ref, s_ref, ss_ref, xcat_ref, xm_ref, xp_ref,
             *, NB, H1, H2, LB, NG, LBu, NGu):
    M = NB * PIMG
    WL = xcat_ref.shape[1] - 2 * LB
    UO2 = wt_ref.shape[1]
    UOg = UO2 // 2

    xcat_ref[...] = jnp.zeros_like(xcat_ref)

    x1 = x1_ref[...].reshape(NB * H1, NGu * LBu)
    bt = btw_ref[...]
    y0s, y1s = [], []
    for j in range(NGu):
        r = jnp.dot(x1[:, j * LBu:(j + 1) * LBu], wt_ref[...],
                    preferred_element_type=jnp.float32)
        y0s.append(r[:, :UOg])
        y1s.append(r[:, UOg:])
    y0 = (jnp.concatenate(y0s, axis=1) if NGu > 1 else y0s[0]) + bt
    y1 = (jnp.concatenate(y1s, axis=1) if NGu > 1 else y1s[0]) + bt
    y0 = y0.astype(jnp.bfloat16)
    y1 = y1.astype(jnp.bfloat16)

    inter = (jnp.dot(r0_ref[...], y0, preferred_element_type=jnp.float32)
             + jnp.dot(r1_ref[...], y1, preferred_element_type=jnp.float32))

    x2e = jnp.dot(x2f_ref[...].reshape(NB * H2, e_ref.shape[0]), e_ref[...],
                  preferred_element_type=jnp.float32)
    data = x2e + inter
    for i in range(NB):
        xcat_ref[pl.ds(TM + i * PIMG, H2), LB:LB + WL] = (
            data[i * H2:(i + 1) * H2].astype(xcat_ref.dtype))

    xm_ref[...] = xcat_ref[TM - 1:TM - 1 + M, :]
    xp_ref[...] = xcat_ref[TM + 1:TM + 1 + M, :]

    acc = _banded_windowed_matmuls(xm_ref[...], xcat_ref[TM:TM + M, :],
                                   xp_ref[...], b1_ref, LB, NG)

    rr = jax.lax.broadcasted_iota(jnp.int32, (M, 1), 0) % PIMG
    mask = (rr < H2).astype(jnp.float32)
    accm = acc * mask
    s_ref[...] = jnp.sum(accm, axis=0, keepdims=True)
    ss_ref[...] = jnp.sum(accm * acc, axis=0, keepdims=True)

    o_ref[...] = acc.astype(o_ref.dtype)


def _k2_body(h_ref, b2_ref, sc_ref, sh_ref,
             o_ref, s_ref, ss_ref, xact_ref, xm_ref, xp_ref, *, NB, H2, LB, NG):
    M = NB * PIMG
    Mx = xact_ref.shape[0]
    WL = xact_ref.shape[1] - 2 * LB

    xact_ref[0:TM, :] = jnp.zeros((TM, xact_ref.shape[1]), xact_ref.dtype)
    xact_ref[Mx - TM:Mx, :] = jnp.zeros((TM, xact_ref.shape[1]), xact_ref.dtype)
    xact_ref[:, 0:LB] = jnp.zeros((Mx, LB), xact_ref.dtype)
    xact_ref[:, LB + WL:] = jnp.zeros((Mx, LB), xact_ref.dtype)

    rr = jax.lax.broadcasted_iota(jnp.int32, (M, 1), 0) % PIMG
    valid = rr < H2
    val = jnp.maximum(h_ref[...].astype(jnp.float32) * sc_ref[...] + sh_ref[...],
                      0.0)
    xact_ref[pl.ds(TM, M), LB:LB + WL] = jnp.where(valid, val, 0.0).astype(
        xact_ref.dtype)

    xm_ref[...] = xact_ref[TM - 1:TM - 1 + M, :]
    xp_ref[...] = xact_ref[TM + 1:TM + 1 + M, :]

    acc = _banded_windowed_matmuls(xm_ref[...], xact_ref[TM:TM + M, :],
                                   xp_ref[...], b2_ref, LB, NG)

    mask = valid.astype(jnp.float32)
    accm = acc * mask
    s_ref[...] = jnp.sum(accm, axis=0, keepdims=True)
    ss_ref[...] = jnp.sum(accm * acc, axis=0, keepdims=True)

    for i in range(NB):
        o_ref[i, :, :] = acc[i * PIMG:i * PIMG + H2].astype(o_ref.dtype)


def _k3_body(x_ref, p_ref, sc_ref, sh_ref, o_ref, *, H2, W2):
    xp = jnp.dot(x_ref[...], p_ref[...], preferred_element_type=jnp.float32)
    y = jnp.maximum(xp * sc_ref[...] + sh_ref[...], 0.0)
    NBk, Cout = o_ref.shape[0], o_ref.shape[1]
    y4 = y.reshape(NBk, H2, Cout, W2)
    o_ref[...] = jnp.transpose(y4, (0, 2, 1, 3))


def kernel(up_w, up_b, conv1_w, conv2_w, bn1_gamma, bn1_beta,
           bn2_gamma, bn2_beta, x1_nchw, x2_nchw):
    eps = 1e-5
    x1 = jnp.transpose(x1_nchw, (0, 2, 3, 1)).astype(jnp.float32)
    x2 = jnp.transpose(x2_nchw, (0, 2, 3, 1)).astype(jnp.float32)
    N, H1, W1, Cin = x1.shape
    _, H2, W2, C2 = x2.shape
    Cup = up_w.shape[1]
    Cmid = conv1_w.shape[0]
    Cout = conv2_w.shape[0]
    Ctot = C2 + Cup
    assert H2 == 2 * H1 and W2 == 2 * W1
    assert conv1_w.shape[1] == Ctot
    assert H2 < PIMG and PIMG % 8 == 0

    bf = jnp.bfloat16
    x1f = x1.reshape(N, H1, W1 * Cin).astype(bf)
    x2f = x2.reshape(N, H2, W2 * C2).astype(bf)

    GW1 = min(W2, max(1, 128 // Ctot))
    GW2 = min(W2, max(1, 128 // Cmid))
    GWu = min(W1, max(1, 128 // Cin))
    assert W2 % GW1 == 0 and W2 % GW2 == 0 and W1 % GWu == 0
    LB1, LBo1, NG1 = GW1 * Ctot, GW1 * Cmid, W2 // GW1
    LB2, LBo2, NG2 = GW2 * Cmid, GW2 * Cout, W2 // GW2
    LBu, UOg, NGu = GWu * Cin, 2 * GWu * Ctot, W1 // GWu

    Wt = _pack_convt(up_w, GWu, C2)
    Wt2 = jnp.concatenate([Wt[0], Wt[1]], axis=1).astype(bf)
    btw = jnp.tile(jnp.concatenate([jnp.zeros((C2,), jnp.float32),
                                    up_b.astype(jnp.float32)]), W2)[None, :]
    B1w = _band_windows(conv1_w, GW1).astype(bf)
    B2w = _band_windows(conv2_w, GW2).astype(bf)
    Ex = np.zeros((W2 * C2, W2 * Ctot), np.float32)
    for w in range(W2):
        for c in range(C2):
            Ex[w * C2 + c, w * Ctot + c] = 1.0
    Ex = jnp.asarray(Ex, dtype=bf)

    W1K = W1 * Cin
    WCtot = W2 * Ctot
    WCmid = W2 * Cmid
    WCout = W2 * Cout

    NB = 16 if N % 16 == 0 else (8 if N % 8 == 0 else 1)
    G = N // NB
    M = NB * PIMG
    Mx = M + 2 * TM
    Rm = _interleave_mats(NB, H1)

    k1 = functools.partial(_k1_body, NB=NB, H1=H1, H2=H2,
                           LB=LB1, NG=NG1, LBu=LBu, NGu=NGu)
    h1, s1, ss1 = pl.pallas_call(
        k1,
        out_shape=(
            jax.ShapeDtypeStruct((G * M, WCmid), bf),
            jax.ShapeDtypeStruct((G, 1, WCmid), jnp.float32),
            jax.ShapeDtypeStruct((G, 1, WCmid), jnp.float32),
        ),
        grid=(G,),
        in_specs=[
            pl.BlockSpec((NB, H1, W1K), lambda g: (g, 0, 0)),
            pl.BlockSpec((NB, H2, W2 * C2), lambda g: (g, 0, 0)),
            pl.BlockSpec((LBu, 2 * UOg), lambda g: (0, 0)),
            pl.BlockSpec((1, WCtot), lambda g: (0, 0)),
            pl.BlockSpec((NB * H2, NB * H1), lambda g: (0, 0)),
            pl.BlockSpec((NB * H2, NB * H1), lambda g: (0, 0)),
            pl.BlockSpec((W2 * C2, WCtot), lambda g: (0, 0)),
            pl.BlockSpec((3, 4, 3 * LB1, LBo1), lambda g: (0, 0, 0, 0)),
        ],
        out_specs=(
            pl.BlockSpec((M, WCmid), lambda g: (g, 0)),
            pl.BlockSpec((None, 1, WCmid), lambda g: (g, 0, 0)),
            pl.BlockSpec((None, 1, WCmid), lambda g: (g, 0, 0)),
        ),
        scratch_shapes=[pltpu.VMEM((Mx, WCtot + 2 * LB1), bf),
                        pltpu.VMEM((M, WCtot + 2 * LB1), bf),
                        pltpu.VMEM((M, WCtot + 2 * LB1), bf)],
        compiler_params=pltpu.CompilerParams(dimension_semantics=("parallel",)),
    )(x1f, x2f, Wt2, btw, Rm[0], Rm[1], Ex, B1w)

    scale1, shift1 = _bn_affine(s1, ss1, N * H2 * W2, Cmid, bn1_gamma, bn1_beta, eps)

    k2 = functools.partial(_k2_body, NB=NB, H2=H2, LB=LB2, NG=NG2)
    h2, s2, ss2 = pl.pallas_call(
        k2,
        out_shape=(
            jax.ShapeDtypeStruct((N, H2, WCout), bf),
            jax.ShapeDtypeStruct((G, 1, WCout), jnp.float32),
            jax.ShapeDtypeStruct((G, 1, WCout), jnp.float32),
        ),
        grid=(G,),
        in_specs=[
            pl.BlockSpec((M, WCmid), lambda g: (g, 0)),
            pl.BlockSpec((3, 4, 3 * LB2, LBo2), lambda g: (0, 0, 0, 0)),
            pl.BlockSpec((1, WCmid), lambda g: (0, 0)),
            pl.BlockSpec((1, WCmid), lambda g: (0, 0)),
        ],
        out_specs=(
            pl.BlockSpec((NB, H2, WCout), lambda g: (g, 0, 0)),
            pl.BlockSpec((None, 1, WCout), lambda g: (g, 0, 0)),
            pl.BlockSpec((None, 1, WCout), lambda g: (g, 0, 0)),
        ),
        scratch_shapes=[pltpu.VMEM((Mx, WCmid + 2 * LB2), bf),
                        pltpu.VMEM((M, WCmid + 2 * LB2), bf),
                        pltpu.VMEM((M, WCmid + 2 * LB2), bf)],
        compiler_params=pltpu.CompilerParams(dimension_semantics=("parallel",)),
    )(h1, B2w, jnp.tile(scale1, W2)[None, :], jnp.tile(shift1, W2)[None, :])

    scale2, shift2 = _bn_affine(s2, ss2, N * H2 * W2, Cout, bn2_gamma, bn2_beta, eps)

    Pcw = np.zeros((WCout, WCout), np.float32)
    for w in range(W2):
        for c in range(Cout):
            Pcw[w * Cout + c, c * W2 + w] = 1.0
    Pcw = jnp.asarray(Pcw, dtype=bf)

    Mrows = N * H2
    NBk = 16 if N % 16 == 0 else (8 if N % 8 == 0 else 1)
    k3 = functools.partial(_k3_body, H2=H2, W2=W2)
    out = pl.pallas_call(
        k3,
        out_shape=jax.ShapeDtypeStruct((N, Cout, H2, W2), jnp.float32),
        grid=(N // NBk,),
        in_specs=[
            pl.BlockSpec((NBk * H2, WCout), lambda i: (i, 0)),
            pl.BlockSpec((WCout, WCout), lambda i: (0, 0)),
            pl.BlockSpec((1, WCout), lambda i: (0, 0)),
            pl.BlockSpec((1, WCout), lambda i: (0, 0)),
        ],
        out_specs=pl.BlockSpec((NBk, Cout, H2, W2), lambda i: (i, 0, 0, 0)),
        compiler_params=pltpu.CompilerParams(dimension_semantics=("parallel",)),
    )(h2.reshape(Mrows, WCout), Pcw, jnp.repeat(scale2, W2)[None, :],
      jnp.repeat(shift2, W2)[None, :])
    return out


# speedup vs baseline: 1.3618x; 1.1072x over previous
import functools

import jax
import jax.numpy as jnp
import numpy as np
from jax.experimental import pallas as pl
from jax.experimental.pallas import tpu as pltpu

PIMG = 40
TM = 8


def _pack_convt(up_w, W1, C2):
    Cin, Cup = up_w.shape[0], up_w.shape[1]
    A = jnp.transpose(up_w.astype(jnp.float32), (2, 0, 3, 1))
    A = jnp.concatenate([jnp.zeros((2, Cin, 2, C2), jnp.float32), A], axis=3)
    A = A.reshape(2, Cin, 2 * (C2 + Cup))
    eye = jnp.eye(W1, dtype=jnp.float32)
    Wt = jnp.einsum("wu,hij->hwiuj", eye, A)
    return Wt.reshape(2, W1 * Cin, W1 * 2 * (C2 + Cup))


def _band_windows(w_oihw, GW):
    Cin, Cout = w_oihw.shape[1], w_oihw.shape[0]
    assert GW >= 3
    Wk = jnp.transpose(w_oihw.astype(jnp.float32), (2, 3, 1, 0))
    mats = []
    for kh in range(3):
        m = None
        for kw in range(3):
            S = np.zeros((2 * GW, GW), np.float32)
            for w in range(GW):
                S[w + kw + 1, w] = 1.0
            t = jnp.einsum("vw,io->viwo", jnp.asarray(S), Wk[kh, kw])
            m = t if m is None else m + t
        mats.append(m.reshape(2 * GW * Cin, GW * Cout))
    Bg = jnp.stack(mats)
    mL = jnp.ones((2 * GW * Cin, 1), jnp.float32).at[:2 * Cin].set(0.0)
    mR = jnp.ones((2 * GW * Cin, 1), jnp.float32).at[(GW + 2) * Cin:].set(0.0)
    return jnp.stack([Bg * mL, Bg, Bg * mR, Bg * mL * mR], axis=1)


def _interleave_mats(NB, H1):
    H2 = 2 * H1
    R = np.zeros((2, NB * H2, NB * H1), np.float32)
    for i in range(NB):
        for h in range(H1):
            R[0, i * H2 + 2 * h, i * H1 + h] = 1.0
            R[1, i * H2 + 2 * h + 1, i * H1 + h] = 1.0
    return jnp.asarray(R, dtype=jnp.bfloat16)


def _bn_affine(sum_lane, ssq_lane, count, C, gamma, beta, eps):
    s = jnp.sum(sum_lane.reshape(-1, C), axis=0)
    ss = jnp.sum(ssq_lane.reshape(-1, C), axis=0)
    mean = s / count
    var = ss / count - mean * mean
    scale = gamma / jnp.sqrt(var + eps)
    shift = beta - mean * scale
    return scale, shift


def _banded_windowed_matmuls(xm, x0, xp, b_ref, LB, NG):
    groups = []
    for j in range(NG):
        sl = slice(j * LB, (j + 2) * LB)
        v = 1 if 0 < j < NG - 1 else (3 if NG == 1 else (0 if j == 0 else 2))
        a = jnp.dot(xm[:, sl], b_ref[0, v], preferred_element_type=jnp.float32)
        a = a + jnp.dot(x0[:, sl], b_ref[1, v], preferred_element_type=jnp.float32)
        a = a + jnp.dot(xp[:, sl], b_ref[2, v], preferred_element_type=jnp.float32)
        groups.append(a)
    return jnp.concatenate(groups, axis=1) if len(groups) > 1 else groups[0]


def _k1_body(x1_ref, x2f_ref, wt_ref, btw_ref, r0_ref, r1_ref, e_ref, b1_ref,
             o_ref, s_ref, ss_ref, xcat_ref, xm_ref, xp_ref,
             *, NB, H1, H2, LB, NG, DOFF, LBu, NGu):
    M = NB * PIMG
    WL = xcat_ref.shape[1] - LB
    UO2 = wt_ref.shape[1]
    UOg = UO2 // 2

    xcat_ref[...] = jnp.zeros_like(xcat_ref)

    x1 = x1_ref[...].reshape(NB * H1, NGu * LBu)
    bt = btw_ref[...]
    y0s, y1s = [], []
    for j in range(NGu):
        r = jnp.dot(x1[:, j * LBu:(j + 1) * LBu], wt_ref[...],
                    preferred_element_type=jnp.float32)
        y0s.append(r[:, :UOg])
        y1s.append(r[:, UOg:])
    y0 = (jnp.concatenate(y0s, axis=1) if NGu > 1 else y0s[0]) + bt
    y1 = (jnp.concatenate(y1s, axis=1) if NGu > 1 else y1s[0]) + bt
    y0 = y0.astype(jnp.bfloat16)
    y1 = y1.astype(jnp.bfloat16)

    inter = (jnp.dot(r0_ref[...], y0, preferred_element_type=jnp.float32)
             + jnp.dot(r1_ref[...], y1, preferred_element_type=jnp.float32))

    x2e = jnp.dot(x2f_ref[...].reshape(NB * H2, e_ref.shape[0]), e_ref[...],
                  preferred_element_type=jnp.float32)
    data = x2e + inter
    for i in range(NB):
        xcat_ref[pl.ds(TM + i * PIMG, H2), DOFF:DOFF + WL] = (
            data[i * H2:(i + 1) * H2].astype(xcat_ref.dtype))

    xm_ref[...] = xcat_ref[TM - 1:TM - 1 + M, :]
    xp_ref[...] = xcat_ref[TM + 1:TM + 1 + M, :]

    acc = _banded_windowed_matmuls(xm_ref[...], xcat_ref[TM:TM + M, :],
                                   xp_ref[...], b1_ref, LB, NG)

    rr = jax.lax.broadcasted_iota(jnp.int32, (M, 1), 0) % PIMG
    mask = (rr < H2).astype(jnp.float32)
    accm = acc * mask
    s_ref[...] = jnp.sum(accm, axis=0, keepdims=True)
    ss_ref[...] = jnp.sum(accm * acc, axis=0, keepdims=True)

    o_ref[...] = acc.astype(o_ref.dtype)


def _k2_body(h_ref, b2_ref, sc_ref, sh_ref,
             o_ref, s_ref, ss_ref, xact_ref, xm_ref, xp_ref,
             *, NB, H2, LB, NG, DOFF):
    M = NB * PIMG
    Mx = xact_ref.shape[0]
    WL = xact_ref.shape[1] - LB

    xact_ref[0:TM, :] = jnp.zeros((TM, xact_ref.shape[1]), xact_ref.dtype)
    xact_ref[Mx - TM:Mx, :] = jnp.zeros((TM, xact_ref.shape[1]), xact_ref.dtype)
    xact_ref[:, 0:DOFF] = jnp.zeros((Mx, DOFF), xact_ref.dtype)
    xact_ref[:, DOFF + WL:] = jnp.zeros((Mx, LB - DOFF), xact_ref.dtype)

    rr = jax.lax.broadcasted_iota(jnp.int32, (M, 1), 0) % PIMG
    valid = rr < H2
    val = jnp.maximum(h_ref[...].astype(jnp.float32) * sc_ref[...] + sh_ref[...],
                      0.0)
    xact_ref[pl.ds(TM, M), DOFF:DOFF + WL] = jnp.where(valid, val, 0.0).astype(
        xact_ref.dtype)

    xm_ref[...] = xact_ref[TM - 1:TM - 1 + M, :]
    xp_ref[...] = xact_ref[TM + 1:TM + 1 + M, :]

    acc = _banded_windowed_matmuls(xm_ref[...], xact_ref[TM:TM + M, :],
                                   xp_ref[...], b2_ref, LB, NG)

    mask = valid.astype(jnp.float32)
    accm = acc * mask
    s_ref[...] = jnp.sum(accm, axis=0, keepdims=True)
    ss_ref[...] = jnp.sum(accm * acc, axis=0, keepdims=True)

    for i in range(NB):
        o_ref[i, :, :] = acc[i * PIMG:i * PIMG + H2].astype(o_ref.dtype)


def _k3_body(x_ref, p_ref, sc_ref, sh_ref, o_ref, *, H2, W2):
    xp = jnp.dot(x_ref[...], p_ref[...], preferred_element_type=jnp.float32)
    y = jnp.maximum(xp * sc_ref[...] + sh_ref[...], 0.0)
    NBk, Cout = o_ref.shape[0], o_ref.shape[1]
    y4 = y.reshape(NBk, H2, Cout, W2)
    o_ref[...] = jnp.transpose(y4, (0, 2, 1, 3))


def kernel(up_w, up_b, conv1_w, conv2_w, bn1_gamma, bn1_beta,
           bn2_gamma, bn2_beta, x1_nchw, x2_nchw):
    eps = 1e-5
    x1 = jnp.transpose(x1_nchw, (0, 2, 3, 1)).astype(jnp.float32)
    x2 = jnp.transpose(x2_nchw, (0, 2, 3, 1)).astype(jnp.float32)
    N, H1, W1, Cin = x1.shape
    _, H2, W2, C2 = x2.shape
    Cup = up_w.shape[1]
    Cmid = conv1_w.shape[0]
    Cout = conv2_w.shape[0]
    Ctot = C2 + Cup
    assert H2 == 2 * H1 and W2 == 2 * W1
    assert conv1_w.shape[1] == Ctot
    assert H2 < PIMG and PIMG % 8 == 0

    bf = jnp.bfloat16
    x1f = x1.reshape(N, H1, W1 * Cin).astype(bf)
    x2f = x2.reshape(N, H2, W2 * C2).astype(bf)

    GW1 = min(W2, max(1, 128 // Ctot))
    GW2 = min(W2, max(1, 128 // Cmid))
    GWu = min(W1, max(1, 128 // Cin))
    assert W2 % GW1 == 0 and W2 % GW2 == 0 and W1 % GWu == 0
    LB1, LBo1, NG1 = GW1 * Ctot, GW1 * Cmid, W2 // GW1
    LB2, LBo2, NG2 = GW2 * Cmid, GW2 * Cout, W2 // GW2
    LBu, UOg, NGu = GWu * Cin, 2 * GWu * Ctot, W1 // GWu

    Wt = _pack_convt(up_w, GWu, C2)
    Wt2 = jnp.concatenate([Wt[0], Wt[1]], axis=1).astype(bf)
    btw = jnp.tile(jnp.concatenate([jnp.zeros((C2,), jnp.float32),
                                    up_b.astype(jnp.float32)]), W2)[None, :]
    B1w = _band_windows(conv1_w, GW1).astype(bf)
    B2w = _band_windows(conv2_w, GW2).astype(bf)
    Ex = np.zeros((W2 * C2, W2 * Ctot), np.float32)
    for w in range(W2):
        for c in range(C2):
            Ex[w * C2 + c, w * Ctot + c] = 1.0
    Ex = jnp.asarray(Ex, dtype=bf)

    W1K = W1 * Cin
    WCtot = W2 * Ctot
    WCmid = W2 * Cmid
    WCout = W2 * Cout

    NB = 16 if N % 16 == 0 else (8 if N % 8 == 0 else 1)
    G = N // NB
    M = NB * PIMG
    Mx = M + 2 * TM
    Rm = _interleave_mats(NB, H1)

    k1 = functools.partial(_k1_body, NB=NB, H1=H1, H2=H2,
                           LB=LB1, NG=NG1, DOFF=2 * Ctot, LBu=LBu, NGu=NGu)
    h1, s1, ss1 = pl.pallas_call(
        k1,
        out_shape=(
            jax.ShapeDtypeStruct((G * M, WCmid), bf),
            jax.ShapeDtypeStruct((G, 1, WCmid), jnp.float32),
            jax.ShapeDtypeStruct((G, 1, WCmid), jnp.float32),
        ),
        grid=(G,),
        in_specs=[
            pl.BlockSpec((NB, H1, W1K), lambda g: (g, 0, 0)),
            pl.BlockSpec((NB, H2, W2 * C2), lambda g: (g, 0, 0)),
            pl.BlockSpec((LBu, 2 * UOg), lambda g: (0, 0)),
            pl.BlockSpec((1, WCtot), lambda g: (0, 0)),
            pl.BlockSpec((NB * H2, NB * H1), lambda g: (0, 0)),
            pl.BlockSpec((NB * H2, NB * H1), lambda g: (0, 0)),
            pl.BlockSpec((W2 * C2, WCtot), lambda g: (0, 0)),
            pl.BlockSpec((3, 4, 2 * LB1, LBo1), lambda g: (0, 0, 0, 0)),
        ],
        out_specs=(
            pl.BlockSpec((M, WCmid), lambda g: (g, 0)),
            pl.BlockSpec((None, 1, WCmid), lambda g: (g, 0, 0)),
            pl.BlockSpec((None, 1, WCmid), lambda g: (g, 0, 0)),
        ),
        scratch_shapes=[pltpu.VMEM((Mx, WCtot + LB1), bf),
                        pltpu.VMEM((M, WCtot + LB1), bf),
                        pltpu.VMEM((M, WCtot + LB1), bf)],
        compiler_params=pltpu.CompilerParams(dimension_semantics=("parallel",)),
    )(x1f, x2f, Wt2, btw, Rm[0], Rm[1], Ex, B1w)

    scale1, shift1 = _bn_affine(s1, ss1, N * H2 * W2, Cmid, bn1_gamma, bn1_beta, eps)

    k2 = functools.partial(_k2_body, NB=NB, H2=H2, LB=LB2, NG=NG2,
                           DOFF=2 * Cmid)
    h2, s2, ss2 = pl.pallas_call(
        k2,
        out_shape=(
            jax.ShapeDtypeStruct((N, H2, WCout), bf),
            jax.ShapeDtypeStruct((G, 1, WCout), jnp.float32),
            jax.ShapeDtypeStruct((G, 1, WCout), jnp.float32),
        ),
        grid=(G,),
        in_specs=[
            pl.BlockSpec((M, WCmid), lambda g: (g, 0)),
            pl.BlockSpec((3, 4, 2 * LB2, LBo2), lambda g: (0, 0, 0, 0)),
            pl.BlockSpec((1, WCmid), lambda g: (0, 0)),
            pl.BlockSpec((1, WCmid), lambda g: (0, 0)),
        ],
        out_specs=(
            pl.BlockSpec((NB, H2, WCout), lambda g: (g, 0, 0)),
            pl.BlockSpec((None, 1, WCout), lambda g: (g, 0, 0)),
            pl.BlockSpec((None, 1, WCout), lambda g: (g, 0, 0)),
        ),
        scratch_shapes=[pltpu.VMEM((Mx, WCmid + LB2), bf),
                        pltpu.VMEM((M, WCmid + LB2), bf),
                        pltpu.VMEM((M, WCmid + LB2), bf)],
        compiler_params=pltpu.CompilerParams(dimension_semantics=("parallel",)),
    )(h1, B2w, jnp.tile(scale1, W2)[None, :], jnp.tile(shift1, W2)[None, :])

    scale2, shift2 = _bn_affine(s2, ss2, N * H2 * W2, Cout, bn2_gamma, bn2_beta, eps)

    Pcw = np.zeros((WCout, WCout), np.float32)
    for w in range(W2):
        for c in range(Cout):
            Pcw[w * Cout + c, c * W2 + w] = 1.0
    Pcw = jnp.asarray(Pcw, dtype=bf)

    Mrows = N * H2
    NBk = 16 if N % 16 == 0 else (8 if N % 8 == 0 else 1)
    k3 = functools.partial(_k3_body, H2=H2, W2=W2)
    out = pl.pallas_call(
        k3,
        out_shape=jax.ShapeDtypeStruct((N, Cout, H2, W2), jnp.float32),
        grid=(N // NBk,),
        in_specs=[
            pl.BlockSpec((NBk * H2, WCout), lambda i: (i, 0)),
            pl.BlockSpec((WCout, WCout), lambda i: (0, 0)),
            pl.BlockSpec((1, WCout), lambda i: (0, 0)),
            pl.BlockSpec((1, WCout), lambda i: (0, 0)),
        ],
        out_specs=pl.BlockSpec((NBk, Cout, H2, W2), lambda i: (i, 0, 0, 0)),
        compiler_params=pltpu.CompilerParams(dimension_semantics=("parallel",)),
    )(h2.reshape(Mrows, WCout), Pcw, jnp.repeat(scale2, W2)[None, :],
      jnp.repeat(shift2, W2)[None, :])
    return out
